# baseline (speedup 1.0000x reference)
import functools

import jax
import jax.numpy as jnp
import numpy as np
from jax import lax
from jax.experimental import pallas as pl
from jax.experimental.pallas import tpu as pltpu
from jax.experimental.pallas import tpu_sc as plsc

VOCAB = 50001
D_MODEL = 128
B = 4
L = 2048

NC = 2
NS = 16
NW = NC * NS
N_ROWS = B * L
ROWS_PER_W = N_ROWS // NW
CHUNK = 64
N_CHUNKS = ROWS_PER_W // CHUNK
PE_CHUNKS_PER_BATCH = L // ROWS_PER_W


def _pos_encoding(seq_len, d_model):
    pos = jnp.arange(seq_len, dtype=jnp.float32)[:, None]
    div = jnp.exp(jnp.arange(0, d_model, 2, dtype=jnp.float32)
                  * (-np.log(10000.0) / d_model))
    pe = jnp.zeros((seq_len, d_model), dtype=jnp.float32)
    pe = pe.at[:, 0::2].set(jnp.sin(pos * div))
    pe = pe.at[:, 1::2].set(jnp.cos(pos * div))
    return pe


@functools.partial(
    pl.kernel,
    out_type=jax.ShapeDtypeStruct((N_ROWS, D_MODEL), jnp.float32),
    mesh=plsc.VectorSubcoreMesh(core_axis_name="c", subcore_axis_name="s"),
    scratch_types=[
        pltpu.VMEM((N_CHUNKS, CHUNK), jnp.int32),
        pltpu.VMEM((ROWS_PER_W, D_MODEL), jnp.float32),
        pltpu.VMEM_SHARED((L, D_MODEL), jnp.float32),
        pltpu.SemaphoreType.DMA,
        pltpu.SemaphoreType.DMA((N_CHUNKS,)),
        pltpu.SemaphoreType.DMA((N_CHUNKS,)),
        pltpu.SemaphoreType.DMA((N_CHUNKS,)),
    ],
)
def _sc_embed(x_hbm, pe_hbm, table_hbm, out_hbm, idx_v, rows_v, pe_sh,
              isem, psems, gsems, ssems):
    s = lax.axis_index("s")
    w = s * NC + lax.axis_index("c")
    b = w // PE_CHUNKS_PER_BATCH
    col_base = (w % PE_CHUNKS_PER_BATCH) * ROWS_PER_W
    idx_cps = [
        pltpu.async_copy(x_hbm.at[b, pl.ds(col_base + j * CHUNK, CHUNK)],
                         idx_v.at[j], isem)
        for j in range(N_CHUNKS)
    ]
    @pl.when(s == 0)
    def _():
        pltpu.sync_copy(pe_hbm, pe_sh)

    plsc.subcore_barrier()
    pe_cps = [
        pltpu.async_copy(pe_sh.at[pl.ds(col_base + j * CHUNK, CHUNK)],
                         rows_v.at[pl.ds(j * CHUNK, CHUNK)], psems.at[j])
        for j in range(N_CHUNKS)
    ]
    for cp in idx_cps:
        cp.wait()
    g_cps = []
    for j in range(N_CHUNKS):
        pe_cps[j].wait()
        g_cps.append(
            pltpu.async_copy(table_hbm.at[idx_v.at[j]],
                             rows_v.at[pl.ds(j * CHUNK, CHUNK)],
                             gsems.at[j], add=True))
    s_cps = []
    for j in range(N_CHUNKS):
        g_cps[j].wait()
        s_cps.append(
            pltpu.async_copy(rows_v.at[pl.ds(j * CHUNK, CHUNK)],
                             out_hbm.at[pl.ds(w * ROWS_PER_W + j * CHUNK,
                                              CHUNK)], ssems.at[j]))
    for cp in s_cps:
        cp.wait()


def kernel(x, tgt, emb_src, emb_tgt):
    del tgt, emb_tgt
    pe = _pos_encoding(L, D_MODEL)
    out = _sc_embed(x, pe, emb_src)
    return out.reshape(B, L, D_MODEL)


# speedup vs baseline: 1.0333x; 1.0333x over previous
import functools

import jax
import jax.numpy as jnp
import numpy as np
from jax import lax
from jax.experimental import pallas as pl
from jax.experimental.pallas import tpu as pltpu
from jax.experimental.pallas import tpu_sc as plsc

VOCAB = 50001
D_MODEL = 128
B = 4
L = 2048

NC = 2
NS = 16
NW = NC * NS
N_ROWS = B * L
CHUNK = L // NW
N_LANE_SL = D_MODEL // 16


def _pos_encoding(seq_len, d_model):
    pos = jnp.arange(seq_len, dtype=jnp.float32)[:, None]
    div = jnp.exp(jnp.arange(0, d_model, 2, dtype=jnp.float32)
                  * (-np.log(10000.0) / d_model))
    pe = jnp.zeros((seq_len, d_model), dtype=jnp.float32)
    pe = pe.at[:, 0::2].set(jnp.sin(pos * div))
    pe = pe.at[:, 1::2].set(jnp.cos(pos * div))
    return pe


@functools.partial(
    pl.kernel,
    out_type=jax.ShapeDtypeStruct((N_ROWS, D_MODEL), jnp.float32),
    mesh=plsc.VectorSubcoreMesh(core_axis_name="c", subcore_axis_name="s"),
    scratch_types=[
        pltpu.VMEM((B, CHUNK), jnp.int32),
        pltpu.VMEM((B * CHUNK, D_MODEL), jnp.float32),
        pltpu.VMEM((CHUNK, D_MODEL), jnp.float32),
        pltpu.SemaphoreType.DMA,
        pltpu.SemaphoreType.DMA,
        pltpu.SemaphoreType.DMA((B,)),
        pltpu.SemaphoreType.DMA((B,)),
    ],
)
def _sc_embed(x_hbm, pe_hbm, table_hbm, out_hbm, idx_v, rows_v, pe_v,
              isem, psem, gsems, ssems):
    w = lax.axis_index("s") * NC + lax.axis_index("c")
    col = w * CHUNK
    idx_cps = [
        pltpu.async_copy(x_hbm.at[j, pl.ds(col, CHUNK)], idx_v.at[j], isem)
        for j in range(B)
    ]
    pe_cp = pltpu.async_copy(pe_hbm.at[pl.ds(col, CHUNK)], pe_v, psem)
    for cp in idx_cps:
        cp.wait()
    g_cps = [
        pltpu.async_copy(table_hbm.at[idx_v.at[j]],
                         rows_v.at[pl.ds(j * CHUNK, CHUNK)], gsems.at[j])
        for j in range(B)
    ]
    pe_cp.wait()
    s_cps = []
    for j in range(B):
        g_cps[j].wait()

        def add_row(r, carry):
            for t in range(N_LANE_SL):
                sl = pl.ds(t * 16, 16)
                rows_v[j * CHUNK + r, sl] = (rows_v[j * CHUNK + r, sl]
                                             + pe_v[r, sl])
            return carry

        lax.fori_loop(0, CHUNK, add_row, 0)
        s_cps.append(
            pltpu.async_copy(rows_v.at[pl.ds(j * CHUNK, CHUNK)],
                             out_hbm.at[pl.ds(j * L + col, CHUNK)],
                             ssems.at[j]))
    for cp in s_cps:
        cp.wait()


def kernel(x, tgt, emb_src, emb_tgt):
    del tgt, emb_tgt
    pe = _pos_encoding(L, D_MODEL)
    out = _sc_embed(x, pe, emb_src)
    return out.reshape(B, L, D_MODEL)


# speedup vs baseline: 1.0367x; 1.0033x over previous
import functools

import jax
import jax.numpy as jnp
import numpy as np
from jax import lax
from jax.experimental import pallas as pl
from jax.experimental.pallas import tpu as pltpu
from jax.experimental.pallas import tpu_sc as plsc

VOCAB = 50001
D_MODEL = 128
B = 4
L = 2048

NC = 2
NS = 16
NW = NC * NS
N_ROWS = B * L
CHUNK = L // NW
N_LANE_SL = D_MODEL // 16


def _pos_encoding(seq_len, d_model):
    pos = jnp.arange(seq_len, dtype=jnp.float32)[:, None]
    div = jnp.exp(jnp.arange(0, d_model, 2, dtype=jnp.float32)
                  * (-np.log(10000.0) / d_model))
    pe = jnp.zeros((seq_len, d_model), dtype=jnp.float32)
    pe = pe.at[:, 0::2].set(jnp.sin(pos * div))
    pe = pe.at[:, 1::2].set(jnp.cos(pos * div))
    return pe


@functools.partial(
    pl.kernel,
    out_type=jax.ShapeDtypeStruct((N_ROWS, D_MODEL), jnp.float32),
    mesh=plsc.VectorSubcoreMesh(core_axis_name="c", subcore_axis_name="s"),
    scratch_types=[
        pltpu.VMEM((B, CHUNK), jnp.int32),
        pltpu.VMEM((B * CHUNK, D_MODEL), jnp.float32),
        pltpu.VMEM((CHUNK, D_MODEL), jnp.float32),
        pltpu.SemaphoreType.DMA((B,)),
        pltpu.SemaphoreType.DMA,
        pltpu.SemaphoreType.DMA((B,)),
        pltpu.SemaphoreType.DMA((2 * B,)),
    ],
)
def _sc_embed(x_hbm, pe_hbm, table_hbm, out_hbm, idx_v, rows_v, pe_v,
              isems, psem, gsems, ssems):
    w = lax.axis_index("s") * NC + lax.axis_index("c")
    col = w * CHUNK
    HALF = CHUNK // 2
    idx_cps = [
        pltpu.async_copy(x_hbm.at[j, pl.ds(col, CHUNK)], idx_v.at[j],
                         isems.at[j])
        for j in range(B)
    ]
    pe_cp = pltpu.async_copy(pe_hbm.at[pl.ds(col, CHUNK)], pe_v, psem)
    g_cps = []
    for j in range(B):
        idx_cps[j].wait()
        g_cps.append(
            pltpu.async_copy(table_hbm.at[idx_v.at[j]],
                             rows_v.at[pl.ds(j * CHUNK, CHUNK)],
                             gsems.at[j]))
    pe_cp.wait()
    s_cps = []
    for j in range(B):
        g_cps[j].wait()
        for h in range(2):

            def add_row(r, carry, j=j, h=h):
                for t in range(N_LANE_SL):
                    sl = pl.ds(t * 16, 16)
                    rows_v[j * CHUNK + h * HALF + r, sl] = (
                        rows_v[j * CHUNK + h * HALF + r, sl]
                        + pe_v[h * HALF + r, sl])
                return carry

            lax.fori_loop(0, HALF, add_row, 0)
            s_cps.append(
                pltpu.async_copy(
                    rows_v.at[pl.ds(j * CHUNK + h * HALF, HALF)],
                    out_hbm.at[pl.ds(j * L + col + h * HALF, HALF)],
                    ssems.at[2 * j + h]))
    for cp in s_cps:
        cp.wait()


def kernel(x, tgt, emb_src, emb_tgt):
    del tgt, emb_tgt
    pe = _pos_encoding(L, D_MODEL)
    out = _sc_embed(x, pe, emb_src)
    return out.reshape(B, L, D_MODEL)
